# Initial kernel scaffold; baseline (speedup 1.0000x reference)
import functools

import jax
import jax.numpy as jnp
from jax import lax
from jax.experimental import pallas as pl
from jax.experimental.pallas import tpu as pltpu
from jax.experimental.pallas import tpu_sc as plsc

B = 128
NPG = 64
E = 262144
D = 16
FULL_E = B * NPG * NPG

NC = 2
NS = 16
LANES = 16

NCHUNK = 8
CH_ROWS = FULL_E // NCHUNK
CH_SHIFT = 16
DUMMY = 256
EPT = E // NS
GROUPS = EPT // LANES
KPC = NCHUNK // NC
CAP = 4096
CAPF = CAP + LANES
BATCH = 128
NB = CAP // BATCH
ZROWS = (CH_ROWS + DUMMY) // NS // 4

_mesh = plsc.VectorSubcoreMesh(core_axis_name="c", subcore_axis_name="s")


@functools.partial(
    pl.kernel,
    out_type=jax.ShapeDtypeStruct((FULL_E, D), jnp.float32),
    mesh=_mesh,
    scratch_types=[
        pltpu.VMEM((EPT,), jnp.int32),
        pltpu.VMEM((EPT,), jnp.int32),
        pltpu.VMEM((KPC, CAPF), jnp.int32),
        pltpu.VMEM((KPC, CAPF), jnp.int32),
        pltpu.VMEM((NB, BATCH), jnp.int32),
        pltpu.VMEM((NB, BATCH), jnp.int32),
        pltpu.VMEM((BATCH, D), jnp.float32),
        pltpu.VMEM((ZROWS, D), jnp.float32),
        pltpu.VMEM_SHARED((CH_ROWS + DUMMY, D), jnp.float32),
        pltpu.SemaphoreType.DMA,
    ],
)
def _sc_scatter_add(u_hbm, v_hbm, attr_hbm, out_hbm,
                    ubuf, vbuf, elist, rlist, e2d, r2d, rows, zbuf, acc, sem):
    c = lax.axis_index("c")
    s = lax.axis_index("s")
    iota = lax.broadcasted_iota(jnp.int32, (LANES,), 0)
    zero16f = jnp.zeros((LANES,), jnp.float32)

    def _zb(i, carry):
        zbuf[i, :] = zero16f
        return carry
    lax.fori_loop(0, ZROWS, _zb, 0)

    def _pf(i, carry):
        off = i * LANES
        dspread = CH_ROWS + ((off + iota) & (DUMMY - 1))
        zero16i = jnp.zeros((LANES,), jnp.int32)
        for k in range(KPC):
            rlist[k, pl.ds(off, LANES)] = dspread
            elist[k, pl.ds(off, LANES)] = zero16i
        return carry
    lax.fori_loop(0, CAPF // LANES, _pf, 0)

    ebase = s * EPT
    pltpu.sync_copy(u_hbm.at[pl.ds(ebase, EPT)], ubuf)
    pltpu.sync_copy(v_hbm.at[pl.ds(ebase, EPT)], vbuf)

    def _scan(m, cnts):
        off = m * LANES
        u16 = ubuf[pl.ds(off, LANES)]
        v16 = vbuf[pl.ds(off, LANES)]
        slot = (u16 << 6) | (v16 & 63)
        r = slot & (CH_ROWS - 1)
        chunkv = slot >> CH_SHIFT
        eid = ebase + off + iota
        new = []
        for k in range(KPC):
            cnt = cnts[k]
            mk = chunkv == (k * NC + c)
            plsc.store_compressed(elist.at[k, pl.ds(cnt, LANES)], eid, mask=mk)
            plsc.store_compressed(rlist.at[k, pl.ds(cnt, LANES)], r, mask=mk)
            pc = plsc.all_reduce_population_count(mk)
            if pc.ndim:
                pc = pc[0]
            new.append(cnt + pc)
        return tuple(new)
    zero = jnp.zeros((), jnp.int32)
    cnts = lax.fori_loop(0, GROUPS, _scan, (zero, zero, zero, zero))

    zstripe = (CH_ROWS + DUMMY) // NS
    orows = CH_ROWS // NS
    for k in range(KPC):
        chunk_id = k * NC + c
        for z in range(zstripe // ZROWS):
            pltpu.sync_copy(zbuf, acc.at[pl.ds(s * zstripe + z * ZROWS, ZROWS), :])
        plsc.subcore_barrier()
        def _cp(i, carry):
            row = i >> 3
            colg = (i & 7) << 4
            src_off = i << 4
            e2d[row, pl.ds(colg, LANES)] = elist[k, pl.ds(src_off, LANES)]
            r2d[row, pl.ds(colg, LANES)] = rlist[k, pl.ds(src_off, LANES)]
            return carry
        lax.fori_loop(0, CAP // LANES, _cp, 0)
        nb = (cnts[k] + (BATCH - 1)) >> 7
        def _gs(j, carry):
            pltpu.async_copy(attr_hbm.at[e2d.at[j]], rows, sem).wait()
            pltpu.sync_copy(rows, acc.at[r2d.at[j]], add=True)
            return carry
        lax.fori_loop(0, nb, _gs, 0)
        plsc.subcore_barrier()
        pltpu.sync_copy(
            acc.at[pl.ds(s * orows, orows), :],
            out_hbm.at[pl.ds(chunk_id * CH_ROWS + s * orows, orows), :])
        plsc.subcore_barrier()


_FCOLS = 4096


def _full_idx_body(o_ref):
    i = pl.program_id(0)
    col = i * _FCOLS + lax.broadcasted_iota(jnp.int32, (2, _FCOLS), 1)
    rowsel = lax.broadcasted_iota(jnp.int32, (2, _FCOLS), 0)
    src = col >> 6
    dst = ((col >> 12) << 6) | (col & 63)
    o_ref[...] = jnp.where(rowsel == 0, src, dst)


def _full_idx():
    return pl.pallas_call(
        _full_idx_body,
        out_shape=jax.ShapeDtypeStruct((2, FULL_E), jnp.int32),
        grid=(FULL_E // _FCOLS,),
        out_specs=pl.BlockSpec((2, _FCOLS), lambda i: (0, i)),
    )()


def kernel(edge_index, edge_attr, batch_vec):
    u = edge_index[0]
    v = edge_index[1]
    out_val = _sc_scatter_add(u, v, edge_attr)
    full_idx = _full_idx()
    return full_idx, out_val


# speedup vs baseline: 1.6737x; 1.6737x over previous
import functools

import jax
import jax.numpy as jnp
from jax import lax
from jax.experimental import pallas as pl
from jax.experimental.pallas import tpu as pltpu
from jax.experimental.pallas import tpu_sc as plsc

B = 128
NPG = 64
E = 262144
D = 16
FULL_E = B * NPG * NPG

NC = 2
NS = 16
LANES = 16

NCHUNK = 16
CH_ROWS = FULL_E // NCHUNK
CH_SHIFT = 15
DUMMY = 256
EPT = E // NS
GROUPS = EPT // LANES
KPC = NCHUNK // NC
CAP = 2048
CAPF = CAP + LANES
BATCH = 128
NB = CAP // BATCH
ZROWS = (CH_ROWS + DUMMY) // NS // 2

_mesh = plsc.VectorSubcoreMesh(core_axis_name="c", subcore_axis_name="s")


@functools.partial(
    pl.kernel,
    out_type=jax.ShapeDtypeStruct((FULL_E, D), jnp.float32),
    mesh=_mesh,
    compiler_params=pltpu.CompilerParams(
        use_tc_tiling_on_sc=False, needs_layout_passes=False),
    scratch_types=[
        pltpu.VMEM((EPT,), jnp.int32),
        pltpu.VMEM((EPT,), jnp.int32),
        *[pltpu.VMEM((CAPF,), jnp.int32) for _ in range(KPC)],
        *[pltpu.VMEM((CAPF,), jnp.int32) for _ in range(KPC)],
        pltpu.VMEM((NB, BATCH), jnp.int32),
        pltpu.VMEM((NB, BATCH), jnp.int32),
        pltpu.VMEM((BATCH, D), jnp.float32),
        pltpu.VMEM((ZROWS, D), jnp.float32),
        pltpu.VMEM_SHARED((CH_ROWS + DUMMY, D), jnp.float32),
        pltpu.SemaphoreType.DMA,
    ],
)
def _sc_scatter_add(u_hbm, v_hbm, attr_hbm, out_hbm,
                    ubuf, vbuf, *rest):
    elist, rlist = rest[:KPC], rest[KPC:2 * KPC]
    e2d, r2d, rows, zbuf, acc, sem = rest[2 * KPC:]
    c = lax.axis_index("c")
    s = lax.axis_index("s")
    iota = lax.broadcasted_iota(jnp.int32, (LANES,), 0)
    ones16i = jnp.ones((LANES,), jnp.int32)
    zero16i = jnp.zeros((LANES,), jnp.int32)
    zero16f = jnp.zeros((LANES,), jnp.float32)

    def _zb(i, carry):
        zbuf[i, :] = zero16f
        return carry
    lax.fori_loop(0, ZROWS, _zb, 0)

    def _pf(i, carry):
        off = i * LANES
        dspread = CH_ROWS + ((off + iota) & (DUMMY - 1))
        for k in range(KPC):
            rlist[k][pl.ds(off, LANES)] = dspread
            elist[k][pl.ds(off, LANES)] = zero16i
        return carry
    lax.fori_loop(0, CAPF // LANES, _pf, 0)

    ebase = s * EPT
    pltpu.sync_copy(u_hbm.at[pl.ds(ebase, EPT)], ubuf)
    pltpu.sync_copy(v_hbm.at[pl.ds(ebase, EPT)], vbuf)

    def _scan(m, cnts):
        off = m * LANES
        u16 = ubuf[pl.ds(off, LANES)]
        v16 = vbuf[pl.ds(off, LANES)]
        slot = (u16 << 6) | (v16 & 63)
        r = slot & (CH_ROWS - 1)
        chunkv = slot >> CH_SHIFT
        eid = ebase + off + iota
        new = []
        for k in range(KPC):
            cnt = cnts[k]
            mk = chunkv == (k * NC + c)
            mi = jnp.where(mk, ones16i, zero16i)
            pres = plsc.cumsum(mi)
            dest = jnp.where(mk, cnt + pres - 1, CAP + iota)
            plsc.store_scatter(elist[k], [dest], eid)
            plsc.store_scatter(rlist[k], [dest], r)
            new.append(cnt + jnp.sum(mi))
        return tuple(new)
    zero = jnp.zeros((), jnp.int32)
    cnts = lax.fori_loop(0, GROUPS, _scan, (zero,) * KPC)

    zstripe = (CH_ROWS + DUMMY) // NS
    orows = CH_ROWS // NS
    for k in range(KPC):
        chunk_id = k * NC + c
        for z in range(zstripe // ZROWS):
            pltpu.sync_copy(zbuf, acc.at[pl.ds(s * zstripe + z * ZROWS, ZROWS), :])
        plsc.subcore_barrier()
        def _cp(i, carry):
            row = i >> 3
            colg = (i & 7) << 4
            src_off = i << 4
            e2d[row, pl.ds(colg, LANES)] = elist[k][pl.ds(src_off, LANES)]
            r2d[row, pl.ds(colg, LANES)] = rlist[k][pl.ds(src_off, LANES)]
            return carry
        lax.fori_loop(0, CAP // LANES, _cp, 0)
        nb = (cnts[k] + (BATCH - 1)) >> 7
        def _gs(j, carry):
            pltpu.async_copy(attr_hbm.at[e2d.at[j]], rows, sem).wait()
            pltpu.sync_copy(rows, acc.at[r2d.at[j]], add=True)
            return carry
        lax.fori_loop(0, nb, _gs, 0)
        plsc.subcore_barrier()
        pltpu.sync_copy(
            acc.at[pl.ds(s * orows, orows), :],
            out_hbm.at[pl.ds(chunk_id * CH_ROWS + s * orows, orows), :])
        plsc.subcore_barrier()


_FCOLS = 4096


def _full_idx_body(o_ref):
    i = pl.program_id(0)
    col = i * _FCOLS + lax.broadcasted_iota(jnp.int32, (2, _FCOLS), 1)
    rowsel = lax.broadcasted_iota(jnp.int32, (2, _FCOLS), 0)
    src = col >> 6
    dst = ((col >> 12) << 6) | (col & 63)
    o_ref[...] = jnp.where(rowsel == 0, src, dst)


def _full_idx():
    return pl.pallas_call(
        _full_idx_body,
        out_shape=jax.ShapeDtypeStruct((2, FULL_E), jnp.int32),
        grid=(FULL_E // _FCOLS,),
        out_specs=pl.BlockSpec((2, _FCOLS), lambda i: (0, i)),
    )()


def kernel(edge_index, edge_attr, batch_vec):
    u = edge_index[0]
    v = edge_index[1]
    out_val = _sc_scatter_add(u, v, edge_attr)
    full_idx = _full_idx()
    return full_idx, out_val
